# baseline (speedup 1.0000x reference)
import functools
import jax
import jax.numpy as jnp
from jax import lax
from jax.experimental import pallas as pl
from jax.experimental.pallas import tpu as pltpu

K_SEL = 1024
N_PTS = 16384
NB = 2048
GRID = N_PTS // NB
IMIN = -2147483648


def _scores_body(x_ref, W1_ref, b1_ref, W2_ref, b2_ref, W3_ref, b3_ref,
                 wlw_ref, wlb_ref, key_ref, thr_ref):
    i = pl.program_id(0)
    W1 = W1_ref[...]
    b1 = b1_ref[...]
    W2 = W2_ref[...]
    b2 = b2_ref[...]
    W3 = W3_ref[...]
    b3 = b3_ref[...]
    wl = wlw_ref[...]
    wlb = wlb_ref[...]

    s = jnp.zeros((1, NB), jnp.float32)
    for b in range(4):
        xb = x_ref[b]
        h1 = jnp.maximum(
            jnp.dot(W1, xb, preferred_element_type=jnp.float32) + b1, 0.0)
        h2 = jnp.maximum(
            jnp.dot(W2, h1, preferred_element_type=jnp.float32) + b2, 0.0)
        h3 = jnp.dot(W3, h2, preferred_element_type=jnp.float32) + b3
        s = s + (jnp.dot(wl, h3, preferred_element_type=jnp.float32) + wlb)
    s = s * 0.25

    u = lax.bitcast_convert_type(s, jnp.int32)
    key = jnp.where(s >= 0.0, u, u ^ jnp.int32(0x7FFFFFFF))
    key_ref[pl.ds(i, 1), :] = key

    @pl.when(i == GRID - 1)
    def _():
        kall = key_ref[...]

        imin = jnp.int32(IMIN)

        def body(b, pfx):
            cand = pfx | (jnp.int32(1) << (jnp.int32(31) - b))
            t_signed = cand ^ imin
            cnt = jnp.sum((kall >= t_signed).astype(jnp.int32))
            return jnp.where(cnt >= K_SEL, cand, pfx)

        pfx = lax.fori_loop(0, 32, body, jnp.int32(0))
        thr_ref[...] = jnp.broadcast_to(pfx ^ imin, (1, 1))


@jax.jit
def _scores_tc(src_pts, W1, b1, W2, b2, W3, b3, wl_w, wl_b):
    keys, thr = pl.pallas_call(
        _scores_body,
        grid=(GRID,),
        in_specs=[
            pl.BlockSpec((4, 6, NB), lambda i: (0, 0, i)),
            pl.BlockSpec((64, 6), lambda i: (0, 0)),
            pl.BlockSpec((64, 1), lambda i: (0, 0)),
            pl.BlockSpec((128, 64), lambda i: (0, 0)),
            pl.BlockSpec((128, 1), lambda i: (0, 0)),
            pl.BlockSpec((32, 128), lambda i: (0, 0)),
            pl.BlockSpec((32, 1), lambda i: (0, 0)),
            pl.BlockSpec((1, 32), lambda i: (0, 0)),
            pl.BlockSpec((1, 1), lambda i: (0, 0)),
        ],
        out_specs=[
            pl.BlockSpec((GRID, NB), lambda i: (0, 0)),
            pl.BlockSpec((1, 1), lambda i: (0, 0)),
        ],
        out_shape=[
            jax.ShapeDtypeStruct((GRID, NB), jnp.int32),
            jax.ShapeDtypeStruct((1, 1), jnp.int32),
        ],
        compiler_params=pltpu.CompilerParams(
            dimension_semantics=("arbitrary",)),
    )(src_pts, W1, b1.reshape(64, 1), W2, b2.reshape(128, 1), W3,
      b3.reshape(32, 1), wl_w.reshape(1, 32), wl_b.reshape(1, 1))
    return keys, thr


def kernel(src_pts, tgt_pts, W1, b1, W2, b2, W3, b3, wl_w, wl_b):
    keys, _thr = _scores_tc(src_pts, W1, b1, W2, b2, W3, b3, wl_w, wl_b)
    kflat = keys.reshape(N_PTS)
    _, idx = lax.top_k(kflat, K_SEL)
    out = jnp.take(src_pts, idx, axis=2)
    return jnp.transpose(out, (0, 2, 1))
